# baseline (speedup 1.0000x reference)
import jax
import jax.numpy as jnp
from jax.experimental import pallas as pl
from jax.experimental.pallas import tpu as pltpu


def _cell_kernel(A_ref, xTn_ref, xTP_ref, hg_ref,
                 w1x_ref, W1h_ref, b1_ref,
                 w2x_ref, W2h_ref, b2_ref,
                 out_ref,
                 AcP_s, App_s, axf_s, axP_s):
    b = pl.program_id(0)
    nb = pl.num_programs(0)
    f32 = jnp.float32
    bf16 = jnp.bfloat16

    @pl.when(b == 0)
    def _():
        n = A_ref.shape[0]
        Ab = A_ref[...].astype(bf16)
        kk = jax.lax.broadcasted_iota(jnp.int32, (n, n), 0)
        jj = jax.lax.broadcasted_iota(jnp.int32, (n, n), 1)
        P = (kk == 2 * jj - (n - 1) * ((2 * jj) // n)).astype(bf16)
        Pt = (jj == 2 * kk - (n - 1) * ((2 * kk) // n)).astype(bf16)
        AcP = jnp.dot(Ab, P, preferred_element_type=f32).astype(bf16)
        AcP_s[...] = AcP
        App = jnp.dot(Pt, AcP, preferred_element_type=f32).astype(bf16)
        App_s[...] = App
        axf_s[...] = jnp.dot(Ab, xTn_ref[...], preferred_element_type=f32)
        axP_s[...] = jnp.dot(App, xTP_ref[...], preferred_element_type=f32)

    nsub = hg_ref.shape[0]
    gw = 8
    n_half, h2 = hg_ref.shape[1], hg_ref.shape[2]
    hdim = h2 // 2
    for g in range(nsub // gw):
        js = [g * gw + t for t in range(gw)]
        hgs = [hg_ref[j] for j in js]
        hbs = [h.astype(bf16) for h in hgs]

        h_P = jnp.concatenate(
            [jnp.concatenate([hb[:, :hdim] for hb in hbs], axis=1),
             jnp.concatenate([hb[:, hdim:] for hb in hbs], axis=1)], axis=0)

        ah_w = jnp.dot(AcP_s[...], h_P, preferred_element_type=f32)

        us, rlos, rhis = [], [], []
        for t in range(gw):
            bj = b * nsub + js[t]
            onehot = (jax.lax.broadcasted_iota(jnp.int32, (nb * nsub, 1), 0)
                      == bj).astype(f32)
            axc = jnp.dot(axf_s[...], onehot, preferred_element_type=f32)
            ah = ah_w[:, t * hdim:(t + 1) * hdim]
            pre1 = jnp.dot(ah.astype(bf16), W1h_ref[...], preferred_element_type=f32)
            pre1 = pre1 + axc * w1x_ref[...] + b1_ref[...]
            s = jax.nn.sigmoid(pre1)
            rh = (s[:n_half, :] * hgs[t]).astype(bf16)
            us.append(s[n_half:, :])
            rlos.append(rh[:, :hdim])
            rhis.append(rh[:, hdim:])

        rh_P = jnp.concatenate(
            [jnp.concatenate(rlos, axis=1),
             jnp.concatenate(rhis, axis=1)], axis=0)

        pP_w = jnp.dot(App_s[...], rh_P, preferred_element_type=f32)

        for t in range(gw):
            bj = b * nsub + js[t]
            onehot = (jax.lax.broadcasted_iota(jnp.int32, (nb * nsub, 1), 0)
                      == bj).astype(f32)
            axcP = jnp.dot(axP_s[...], onehot, preferred_element_type=f32)
            pP = pP_w[:, t * hdim:(t + 1) * hdim]
            cP = jnp.tanh(jnp.dot(pP.astype(bf16), W2h_ref[...],
                                  preferred_element_type=f32)
                          + axcP * w2x_ref[...] + b2_ref[...])
            c = jnp.concatenate([cP[:n_half], cP[n_half:]], axis=1)
            out_ref[js[t]] = c + us[t] * (hgs[t] - c)


@jax.jit
def kernel(inputs, hidden_state, view, W1, b1, W2, b2):
    B, N = inputs.shape
    H = W2.shape[1]
    Nh = N // 2
    H2 = 2 * H

    bf16 = jnp.bfloat16
    perm = jnp.concatenate([jnp.arange(0, N, 2), jnp.arange(1, N, 2)])
    hg = hidden_state.reshape(B, Nh, H2)
    xTn = inputs.T.astype(bf16)
    xTP = xTn[perm]
    w1x = W1[0:1]
    W1h = W1[1:].astype(bf16)
    w2x = W2[0:1]
    W2h = W2[1:].astype(bf16)
    b1r = b1.reshape(1, H2)
    b2r = b2.reshape(1, H)

    def const(shape):
        nzeros = (0,) * len(shape)
        return pl.BlockSpec(shape, lambda b, _z=nzeros: _z)

    nsub = 8

    def half(hgh, xTnh, xTPh, bh):
        return pl.pallas_call(
            _cell_kernel,
            grid=(bh // nsub,),
            in_specs=[
                const((N, N)),
                const((N, bh)), const((N, bh)),
                pl.BlockSpec((nsub, Nh, H2), lambda b: (b, 0, 0)),
                const((1, H2)), const((H, H2)), const((1, H2)),
                const((1, H)), const((H, H)), const((1, H)),
            ],
            out_specs=pl.BlockSpec((nsub, Nh, H2), lambda b: (b, 0, 0)),
            out_shape=jax.ShapeDtypeStruct((bh, Nh, H2), jnp.float32),
            scratch_shapes=[
                pltpu.VMEM((N, N), jnp.bfloat16),
                pltpu.VMEM((N, N), jnp.bfloat16),
                pltpu.VMEM((N, bh), jnp.float32),
                pltpu.VMEM((N, bh), jnp.float32),
            ],
        )(view, xTnh, xTPh, hgh,
          w1x, W1h, b1r, w2x, W2h, b2r)

    Bh = B // 2
    o1 = half(hidden_state[:Bh].reshape(Bh, Nh, H2),
              xTn[:, :Bh], xTP[:, :Bh], Bh)
    o2 = half(hidden_state[Bh:].reshape(Bh, Nh, H2),
              xTn[:, Bh:], xTP[:, Bh:], Bh)
    return jnp.concatenate([o1.reshape(Bh, N * H), o2.reshape(Bh, N * H)], axis=0)


# speedup vs baseline: 1.2116x; 1.2116x over previous
import jax
import jax.numpy as jnp
from jax.experimental import pallas as pl
from jax.experimental.pallas import tpu as pltpu


def _cell_kernel(A_ref, xTn_ref, xTP_ref, hg_ref,
                 w1x_ref, W1h_ref, b1_ref,
                 w2x_ref, W2h_ref, b2_ref,
                 out_ref,
                 AcP_s, App_s, axf_s, axP_s):
    b = pl.program_id(0)
    nb = pl.num_programs(0)
    f32 = jnp.float32
    bf16 = jnp.bfloat16

    @pl.when(b == 0)
    def _():
        n = A_ref.shape[0]
        Ab = A_ref[...].astype(bf16)
        kk = jax.lax.broadcasted_iota(jnp.int32, (n, n), 0)
        jj = jax.lax.broadcasted_iota(jnp.int32, (n, n), 1)
        P = (kk == 2 * jj - (n - 1) * ((2 * jj) // n)).astype(bf16)
        Pt = (jj == 2 * kk - (n - 1) * ((2 * kk) // n)).astype(bf16)
        AcP = jnp.dot(Ab, P, preferred_element_type=f32).astype(bf16)
        AcP_s[...] = AcP
        App = jnp.dot(Pt, AcP, preferred_element_type=f32).astype(bf16)
        App_s[...] = App
        axf_s[...] = jnp.dot(Ab, xTn_ref[...], preferred_element_type=f32)
        axP_s[...] = jnp.dot(App, xTP_ref[...], preferred_element_type=f32)

    nsub = hg_ref.shape[0]
    gw = 4
    n_half, h2 = hg_ref.shape[1], hg_ref.shape[2]
    hdim = h2 // 2
    for g in range(nsub // gw):
        js = [g * gw + t for t in range(gw)]
        hgs = [hg_ref[j] for j in js]
        hbs = [h.astype(bf16) for h in hgs]

        h_P = jnp.concatenate(
            [jnp.concatenate([hb[:, :hdim] for hb in hbs], axis=1),
             jnp.concatenate([hb[:, hdim:] for hb in hbs], axis=1)], axis=0)

        ah_w = jnp.dot(AcP_s[...], h_P, preferred_element_type=f32)

        us, rlos, rhis = [], [], []
        for t in range(gw):
            bj = b * nsub + js[t]
            onehot = (jax.lax.broadcasted_iota(jnp.int32, (nb * nsub, 1), 0)
                      == bj).astype(f32)
            axc = jnp.dot(axf_s[...], onehot, preferred_element_type=f32)
            ah = ah_w[:, t * hdim:(t + 1) * hdim]
            pre1 = jnp.dot(ah.astype(bf16), W1h_ref[...], preferred_element_type=f32)
            pre1 = pre1 + axc * w1x_ref[...] + b1_ref[...]
            s = jax.nn.sigmoid(pre1)
            rh = (s[:n_half, :] * hgs[t]).astype(bf16)
            us.append(s[n_half:, :])
            rlos.append(rh[:, :hdim])
            rhis.append(rh[:, hdim:])

        rh_P = jnp.concatenate(
            [jnp.concatenate(rlos, axis=1),
             jnp.concatenate(rhis, axis=1)], axis=0)

        pP_w = jnp.dot(App_s[...], rh_P, preferred_element_type=f32)

        for t in range(gw):
            bj = b * nsub + js[t]
            onehot = (jax.lax.broadcasted_iota(jnp.int32, (nb * nsub, 1), 0)
                      == bj).astype(f32)
            axcP = jnp.dot(axP_s[...], onehot, preferred_element_type=f32)
            pP = pP_w[:, t * hdim:(t + 1) * hdim]
            cP = jnp.tanh(jnp.dot(pP.astype(bf16), W2h_ref[...],
                                  preferred_element_type=f32)
                          + axcP * w2x_ref[...] + b2_ref[...])
            c = jnp.concatenate([cP[:n_half], cP[n_half:]], axis=1)
            out_ref[js[t]] = c + us[t] * (hgs[t] - c)


@jax.jit
def kernel(inputs, hidden_state, view, W1, b1, W2, b2):
    B, N = inputs.shape
    H = W2.shape[1]
    Nh = N // 2
    H2 = 2 * H

    bf16 = jnp.bfloat16
    perm = jnp.concatenate([jnp.arange(0, N, 2), jnp.arange(1, N, 2)])
    hg = hidden_state.reshape(B, Nh, H2)
    xTn = inputs.T.astype(bf16)
    xTP = xTn[perm]
    w1x = W1[0:1]
    W1h = W1[1:].astype(bf16)
    w2x = W2[0:1]
    W2h = W2[1:].astype(bf16)
    b1r = b1.reshape(1, H2)
    b2r = b2.reshape(1, H)

    def const(shape):
        nzeros = (0,) * len(shape)
        return pl.BlockSpec(shape, lambda b, _z=nzeros: _z)

    nsub = 8
    out = pl.pallas_call(
        _cell_kernel,
        grid=(B // nsub,),
        in_specs=[
            const((N, N)),
            const((N, B)), const((N, B)),
            pl.BlockSpec((nsub, Nh, H2), lambda b: (b, 0, 0)),
            const((1, H2)), const((H, H2)), const((1, H2)),
            const((1, H)), const((H, H)), const((1, H)),
        ],
        out_specs=pl.BlockSpec((nsub, Nh, H2), lambda b: (b, 0, 0)),
        out_shape=jax.ShapeDtypeStruct((B, Nh, H2), jnp.float32),
        scratch_shapes=[
            pltpu.VMEM((N, N), jnp.bfloat16),
            pltpu.VMEM((N, N), jnp.bfloat16),
            pltpu.VMEM((N, B), jnp.float32),
            pltpu.VMEM((N, B), jnp.float32),
        ],
    )(view, xTn, xTP, hg,
      w1x, W1h, b1r, w2x, W2h, b2r)
    return out.reshape(B, N * H)


# speedup vs baseline: 1.2248x; 1.0109x over previous
import jax
import jax.numpy as jnp
from jax.experimental import pallas as pl
from jax.experimental.pallas import tpu as pltpu


def _cell_kernel(A_ref, xTn_ref, xTP_ref, hg_ref,
                 w1x_ref, W1h_ref, b1_ref,
                 w2x_ref, W2h_ref, b2_ref,
                 out_ref,
                 AcP_s, App_s, axf_s, axP_s):
    b = pl.program_id(0)
    nb = pl.num_programs(0)
    f32 = jnp.float32
    bf16 = jnp.bfloat16

    @pl.when(b == 0)
    def _():
        n = A_ref.shape[0]
        Ab = A_ref[...].astype(bf16)
        kk = jax.lax.broadcasted_iota(jnp.int32, (n, n), 0)
        jj = jax.lax.broadcasted_iota(jnp.int32, (n, n), 1)
        P = (kk == 2 * jj - (n - 1) * ((2 * jj) // n)).astype(bf16)
        Pt = (jj == 2 * kk - (n - 1) * ((2 * kk) // n)).astype(bf16)
        AcP = jnp.dot(Ab, P, preferred_element_type=f32).astype(bf16)
        AcP_s[...] = AcP
        App = jnp.dot(Pt, AcP, preferred_element_type=f32).astype(bf16)
        App_s[...] = App
        axf_s[...] = jnp.dot(Ab, xTn_ref[...], preferred_element_type=f32)
        axP_s[...] = jnp.dot(App, xTP_ref[...], preferred_element_type=f32)

    nsub = hg_ref.shape[0]
    gw = 8
    n_half, h2 = hg_ref.shape[1], hg_ref.shape[2]
    hdim = h2 // 2
    for g in range(nsub // gw):
        js = [g * gw + t for t in range(gw)]
        hgs = [hg_ref[j] for j in js]
        hbs = [h.astype(bf16) for h in hgs]

        h_P = jnp.concatenate(
            [jnp.concatenate([hb[:, :hdim] for hb in hbs], axis=1),
             jnp.concatenate([hb[:, hdim:] for hb in hbs], axis=1)], axis=0)

        ah_w = jnp.dot(AcP_s[...], h_P, preferred_element_type=f32)

        us, rlos, rhis = [], [], []
        for t in range(gw):
            bj = b * nsub + js[t]
            onehot = (jax.lax.broadcasted_iota(jnp.int32, (nb * nsub, 1), 0)
                      == bj).astype(f32)
            axc = jnp.dot(axf_s[...], onehot, preferred_element_type=f32)
            ah = ah_w[:, t * hdim:(t + 1) * hdim]
            pre1 = jnp.dot(ah.astype(bf16), W1h_ref[...], preferred_element_type=f32)
            pre1 = pre1 + axc * w1x_ref[...] + b1_ref[...]
            s = jax.nn.sigmoid(pre1)
            rh = (s[:n_half, :] * hgs[t]).astype(bf16)
            us.append(s[n_half:, :])
            rlos.append(rh[:, :hdim])
            rhis.append(rh[:, hdim:])

        rh_P = jnp.concatenate(
            [jnp.concatenate(rlos, axis=1),
             jnp.concatenate(rhis, axis=1)], axis=0)

        pP_w = jnp.dot(App_s[...], rh_P, preferred_element_type=f32)

        for t in range(gw):
            bj = b * nsub + js[t]
            onehot = (jax.lax.broadcasted_iota(jnp.int32, (nb * nsub, 1), 0)
                      == bj).astype(f32)
            axcP = jnp.dot(axP_s[...], onehot, preferred_element_type=f32)
            pP = pP_w[:, t * hdim:(t + 1) * hdim]
            cP = jnp.tanh(jnp.dot(pP.astype(bf16), W2h_ref[...],
                                  preferred_element_type=f32)
                          + axcP * w2x_ref[...] + b2_ref[...])
            c = jnp.concatenate([cP[:n_half], cP[n_half:]], axis=1)
            out_ref[js[t]] = c + us[t] * (hgs[t] - c)


@jax.jit
def kernel(inputs, hidden_state, view, W1, b1, W2, b2):
    B, N = inputs.shape
    H = W2.shape[1]
    Nh = N // 2
    H2 = 2 * H

    bf16 = jnp.bfloat16
    perm = jnp.concatenate([jnp.arange(0, N, 2), jnp.arange(1, N, 2)])
    hg = hidden_state.reshape(B, Nh, H2)
    xTn = inputs.T.astype(bf16)
    xTP = xTn[perm]
    w1x = W1[0:1]
    W1h = W1[1:].astype(bf16)
    w2x = W2[0:1]
    W2h = W2[1:].astype(bf16)
    b1r = b1.reshape(1, H2)
    b2r = b2.reshape(1, H)

    def const(shape):
        nzeros = (0,) * len(shape)
        return pl.BlockSpec(shape, lambda b, _z=nzeros: _z)

    nsub = 8
    out = pl.pallas_call(
        _cell_kernel,
        grid=(B // nsub,),
        in_specs=[
            const((N, N)),
            const((N, B)), const((N, B)),
            pl.BlockSpec((nsub, Nh, H2), lambda b: (b, 0, 0)),
            const((1, H2)), const((H, H2)), const((1, H2)),
            const((1, H)), const((H, H)), const((1, H)),
        ],
        out_specs=pl.BlockSpec((nsub, Nh, H2), lambda b: (b, 0, 0)),
        out_shape=jax.ShapeDtypeStruct((B, Nh, H2), jnp.float32),
        scratch_shapes=[
            pltpu.VMEM((N, N), jnp.bfloat16),
            pltpu.VMEM((N, N), jnp.bfloat16),
            pltpu.VMEM((N, B), jnp.float32),
            pltpu.VMEM((N, B), jnp.float32),
        ],
    )(view, xTn, xTP, hg,
      w1x, W1h, b1r, w2x, W2h, b2r)
    return out.reshape(B, N * H)


# speedup vs baseline: 1.2766x; 1.0423x over previous
import jax
import jax.numpy as jnp
from jax.experimental import pallas as pl
from jax.experimental.pallas import tpu as pltpu


def _cell_kernel(A_ref, xTn_ref, xTP_ref, hg_ref,
                 w1x_ref, W1h_ref, b1_ref,
                 w2x_ref, W2h_ref, b2_ref,
                 out_ref,
                 AcP_s, App_s, axf_s, axP_s):
    b = pl.program_id(0)
    nb = pl.num_programs(0)
    f32 = jnp.float32
    bf16 = jnp.bfloat16

    @pl.when(b == 0)
    def _():
        n = A_ref.shape[0]
        Ab = A_ref[...].astype(bf16)
        kk = jax.lax.broadcasted_iota(jnp.int32, (n, n), 0)
        jj = jax.lax.broadcasted_iota(jnp.int32, (n, n), 1)
        P = (kk == 2 * jj - (n - 1) * ((2 * jj) // n)).astype(bf16)
        Pt = (jj == 2 * kk - (n - 1) * ((2 * kk) // n)).astype(bf16)
        AcP = jnp.dot(Ab, P, preferred_element_type=f32).astype(bf16)
        AcP_s[...] = AcP
        App = jnp.dot(Pt, AcP, preferred_element_type=f32).astype(bf16)
        App_s[...] = App
        axf_s[...] = jnp.dot(Ab, xTn_ref[...], preferred_element_type=f32)
        axP_s[...] = jnp.dot(App, xTP_ref[...], preferred_element_type=f32)

    nsub = hg_ref.shape[0]
    gw = 8
    n_half, h2 = hg_ref.shape[1], hg_ref.shape[2]
    hdim = h2 // 2
    for g in range(nsub // gw):
        js = [g * gw + t for t in range(gw)]
        hgs = [hg_ref[j] for j in js]
        hbs = [h.astype(bf16) for h in hgs]

        h_P = jnp.concatenate(
            [jnp.concatenate([hb[:, :hdim] for hb in hbs], axis=1),
             jnp.concatenate([hb[:, hdim:] for hb in hbs], axis=1)], axis=0)

        ah_w = jnp.dot(AcP_s[...], h_P, preferred_element_type=f32)

        ii = jax.lax.broadcasted_iota(jnp.int32, (nb * nsub, gw), 0)
        tt = jax.lax.broadcasted_iota(jnp.int32, (nb * nsub, gw), 1)
        onehotM = (ii == b * nsub + g * gw + tt).astype(f32)
        axc_w = jnp.dot(axf_s[...], onehotM, preferred_element_type=f32)
        axcP_w = jnp.dot(axP_s[...], onehotM, preferred_element_type=f32)

        us, rlos, rhis = [], [], []
        for t in range(gw):
            axc = axc_w[:, t:t + 1]
            ah = ah_w[:, t * hdim:(t + 1) * hdim]
            pre1 = jnp.dot(ah.astype(bf16), W1h_ref[...], preferred_element_type=f32)
            pre1 = pre1 + axc * w1x_ref[...] + b1_ref[...]
            s = jax.nn.sigmoid(pre1)
            rh = (s[:n_half, :] * hgs[t]).astype(bf16)
            us.append(s[n_half:, :])
            rlos.append(rh[:, :hdim])
            rhis.append(rh[:, hdim:])

        rh_P = jnp.concatenate(
            [jnp.concatenate(rlos, axis=1),
             jnp.concatenate(rhis, axis=1)], axis=0)

        pP_w = jnp.dot(App_s[...], rh_P, preferred_element_type=f32)

        for t in range(gw):
            axcP = axcP_w[:, t:t + 1]
            pP = pP_w[:, t * hdim:(t + 1) * hdim]
            cP = jnp.tanh(jnp.dot(pP.astype(bf16), W2h_ref[...],
                                  preferred_element_type=f32)
                          + axcP * w2x_ref[...] + b2_ref[...])
            c = jnp.concatenate([cP[:n_half], cP[n_half:]], axis=1)
            out_ref[js[t]] = c + us[t] * (hgs[t] - c)


@jax.jit
def kernel(inputs, hidden_state, view, W1, b1, W2, b2):
    B, N = inputs.shape
    H = W2.shape[1]
    Nh = N // 2
    H2 = 2 * H

    bf16 = jnp.bfloat16
    perm = jnp.concatenate([jnp.arange(0, N, 2), jnp.arange(1, N, 2)])
    hg = hidden_state.reshape(B, Nh, H2)
    xTn = inputs.T.astype(bf16)
    xTP = xTn[perm]
    w1x = W1[0:1]
    W1h = W1[1:].astype(bf16)
    w2x = W2[0:1]
    W2h = W2[1:].astype(bf16)
    b1r = b1.reshape(1, H2)
    b2r = b2.reshape(1, H)

    def const(shape):
        nzeros = (0,) * len(shape)
        return pl.BlockSpec(shape, lambda b, _z=nzeros: _z)

    nsub = 8
    out = pl.pallas_call(
        _cell_kernel,
        grid=(B // nsub,),
        in_specs=[
            const((N, N)),
            const((N, B)), const((N, B)),
            pl.BlockSpec((nsub, Nh, H2), lambda b: (b, 0, 0)),
            const((1, H2)), const((H, H2)), const((1, H2)),
            const((1, H)), const((H, H)), const((1, H)),
        ],
        out_specs=pl.BlockSpec((nsub, Nh, H2), lambda b: (b, 0, 0)),
        out_shape=jax.ShapeDtypeStruct((B, Nh, H2), jnp.float32),
        scratch_shapes=[
            pltpu.VMEM((N, N), jnp.bfloat16),
            pltpu.VMEM((N, N), jnp.bfloat16),
            pltpu.VMEM((N, B), jnp.float32),
            pltpu.VMEM((N, B), jnp.float32),
        ],
    )(view, xTn, xTP, hg,
      w1x, W1h, b1r, w2x, W2h, b2r)
    return out.reshape(B, N * H)
